# baseline (speedup 1.0000x reference)
import functools

import numpy as np

import jax
import jax.numpy as jnp
from jax.experimental import pallas as pl
from jax.experimental.pallas import tpu as pltpu

N = 2048
H = 6
BN = 256


def _layer_kernel(hf_ref, ho_ref, mf_ref, wcat_ref, asrc_ref, adst_ref,
                  out_ref, sum_ref, whp_scr, es_scr, edt_scr, edt2_scr,
                  *, fout, nloc, want_sum):
    wh = jnp.dot(hf_ref[...], wcat_ref[...], preferred_element_type=jnp.float32)
    wh_own = jnp.dot(ho_ref[...], wcat_ref[...],
                     preferred_element_type=jnp.float32)
    es_scr[...] = jnp.dot(wh_own, asrc_ref[...],
                          preferred_element_type=jnp.float32)
    e_dst = jnp.dot(wh, adst_ref[...], preferred_element_type=jnp.float32)
    edt = jnp.transpose(e_dst)
    edt_scr[...] = edt
    edt2_scr[...] = 0.2 * edt
    for h in range(H):
        whp_scr[:, h * 128:h * 128 + fout] = (
            wh[:, h * fout:(h + 1) * fout].astype(jnp.bfloat16))
        whp_scr[:, h * 128 + fout:h * 128 + fout + 1] = jnp.ones(
            (N, 1), jnp.bfloat16)
    edmax = jnp.max(edt, axis=1, keepdims=True)

    def blk(i, _):
        rows = pl.ds(i * BN, BN)
        mfb = mf_ref[rows, :]
        es_blk = es_scr[rows, :]
        for h in range(H):
            esc = es_blk[:, h:h + 1]
            mxc = esc + edmax[h:h + 1, :]
            mx = jnp.maximum(mxc, 0.2 * mxc)
            esm = esc - mx
            esm2 = 0.2 * esc - mx
            e = jnp.maximum(esm + edt_scr[h:h + 1, :],
                            esm2 + edt2_scr[h:h + 1, :])
            p = jnp.exp(e).astype(jnp.bfloat16) * mfb
            of = jnp.dot(p, whp_scr[:, h * 128:h * 128 + fout + 1],
                         preferred_element_type=jnp.float32)
            s = jnp.maximum(of[:, fout:fout + 1], 1e-30)
            ob = of[:, :fout] / s
            ob = jnp.where(ob > 0, ob, jnp.exp(ob) - 1.0)
            out_ref[rows, h * fout:(h + 1) * fout] = ob
        return 0

    jax.lax.fori_loop(0, nloc // BN, blk, 0)
    if want_sum:
        sum_ref[...] = jnp.sum(out_ref[...], axis=0, keepdims=True)


def _layer_call(h_full, h_own, mf_own, wcat, asrc, adst, fout, want_sum=False):
    nloc = h_own.shape[0]
    hf = H * fout
    out_shape = [jax.ShapeDtypeStruct((nloc, hf), jnp.float32),
                 jax.ShapeDtypeStruct((1, hf), jnp.float32)]
    res = pl.pallas_call(
        functools.partial(_layer_kernel, fout=fout, nloc=nloc,
                          want_sum=want_sum),
        out_shape=out_shape,
        scratch_shapes=[
            pltpu.VMEM((N, H * 128), jnp.bfloat16),
            pltpu.VMEM((nloc, H), jnp.float32),
            pltpu.VMEM((H, N), jnp.float32),
            pltpu.VMEM((H, N), jnp.float32),
        ],
    )(h_full, h_own, mf_own, wcat, asrc, adst)
    return res


def _final_kernel(s_ref, wdt_ref, bd_ref, out_ref):
    s = s_ref[...]
    nrm = jnp.maximum(jnp.sqrt(jnp.sum(s * s)), 1e-12)
    sn = s / nrm
    out_ref[...] = jnp.sum(sn * wdt_ref[...], axis=1, keepdims=True) + bd_ref[...]


def _prep(W, a):
    Hh, fin, f = W.shape
    wcat = jnp.transpose(W, (1, 0, 2)).reshape(fin, Hh * f)
    eye = jnp.eye(Hh, dtype=W.dtype)
    asrc = (a[:, :f][:, :, None] * eye[:, None, :]).reshape(Hh * f, Hh)
    adst = (a[:, f:][:, :, None] * eye[:, None, :]).reshape(Hh * f, Hh)
    return wcat, asrc, adst


@jax.jit
def kernel(x, adj, W1, a1, W2, a2, W3, a3, Wd, bd):
    mf = (adj > 0).astype(jnp.bfloat16)
    wc1, as1, ad1 = _prep(W1, a1)
    wc2, as2, ad2 = _prep(W2, a2)
    wc3, as3, ad3 = _prep(W3, a3)
    wdt = jnp.reshape(Wd, (1, 384))
    bd2 = jnp.reshape(bd, (1, 1))

    devs = jax.devices()
    nd = 2 if len(devs) >= 2 and N % (2 * BN) == 0 else 1
    mesh = jax.sharding.Mesh(np.array(devs[:nd]), ("d",))
    P = jax.sharding.PartitionSpec

    def body(x_sh, mf_sh, wc1, as1, ad1, wc2, as2, ad2, wc3, as3, ad3,
             wdt, bd2):
        h_full = jax.lax.all_gather(x_sh, "d", axis=0, tiled=True)
        h1_own, _ = _layer_call(h_full, x_sh, mf_sh, wc1, as1, ad1, 16)
        h1_full = jax.lax.all_gather(h1_own, "d", axis=0, tiled=True)
        h2_own, _ = _layer_call(h1_full, h1_own, mf_sh, wc2, as2, ad2, 32)
        h2_full = jax.lax.all_gather(h2_own, "d", axis=0, tiled=True)
        _, part = _layer_call(h2_full, h2_own, mf_sh, wc3, as3, ad3, 64,
                              want_sum=True)
        s = jax.lax.psum(part, "d")
        y = pl.pallas_call(
            _final_kernel,
            out_shape=jax.ShapeDtypeStruct((1, 1), jnp.float32),
        )(s, wdt, bd2)
        return y

    f = jax.shard_map(
        body, mesh=mesh,
        in_specs=(P("d"), P("d"), P(), P(), P(), P(), P(), P(), P(), P(),
                  P(), P(), P()),
        out_specs=P(),
        check_vma=False,
    )
    y = f(x, mf, wc1, as1, ad1, wc2, as2, ad2, wc3, as3, ad3, wdt, bd2)
    return jnp.reshape(y, (1,))


# speedup vs baseline: 4.0908x; 4.0908x over previous
import functools

import jax
import jax.numpy as jnp
from jax.experimental import pallas as pl
from jax.experimental.pallas import tpu as pltpu

N = 2048
H = 6
BN = 256
LOG2E = 1.4426950408889634


def _layer(xin, wcat_ref, asrc_ref, adst_ref, mf_ref, whp_scr, es_scr,
           edt_scr, edt2_scr, hout_ref, fout):
    wh = jnp.dot(xin, wcat_ref[...], preferred_element_type=jnp.float32)
    es_scr[...] = jnp.dot(wh, asrc_ref[...], preferred_element_type=jnp.float32)
    e_dst = jnp.dot(wh, adst_ref[...], preferred_element_type=jnp.float32)
    edt = jnp.transpose(e_dst)
    edt_scr[...] = LOG2E * edt
    edt2_scr[...] = (0.2 * LOG2E) * edt
    for h in range(H):
        whp_scr[:, h * 128:h * 128 + fout] = (
            wh[:, h * fout:(h + 1) * fout].astype(jnp.bfloat16))
        whp_scr[:, h * 128 + fout:h * 128 + fout + 1] = jnp.ones(
            (N, 1), jnp.bfloat16)
    edmax = jnp.max(edt, axis=1, keepdims=True)

    def blk(i, _):
        rows = pl.ds(i * BN, BN)
        mfb = mf_ref[rows, :]
        es_blk = es_scr[rows, :]
        for h in range(H):
            esc = es_blk[:, h:h + 1]
            mxc = esc + edmax[h:h + 1, :]
            mx = jnp.maximum(mxc, 0.2 * mxc)
            esm = LOG2E * (esc - mx)
            esm2 = LOG2E * (0.2 * esc - mx)
            e = jnp.maximum(esm + edt_scr[h:h + 1, :],
                            esm2 + edt2_scr[h:h + 1, :])
            p = jnp.exp2(e).astype(jnp.bfloat16) * mfb
            of = jnp.dot(p, whp_scr[:, h * 128:h * 128 + fout + 1],
                         preferred_element_type=jnp.float32)
            s = jnp.maximum(of[:, fout:fout + 1], 1e-30)
            ob = of[:, :fout] / s
            ob = jnp.where(ob > 0, ob, jnp.exp(ob) - 1.0)
            hout_ref[rows, h * fout:(h + 1) * fout] = ob
        return 0

    jax.lax.fori_loop(0, N // BN, blk, 0)


def _gnn_kernel(x_ref, mf_ref,
                wc1_ref, as1_ref, ad1_ref,
                wc2_ref, as2_ref, ad2_ref,
                wc3_ref, as3_ref, ad3_ref,
                wdt_ref, bd_ref, out_ref,
                wh_scr, es_scr, edt_scr, edt2_scr, h1_scr, h2_scr, h3_scr):
    _layer(x_ref[...], wc1_ref, as1_ref, ad1_ref, mf_ref, wh_scr, es_scr,
           edt_scr, edt2_scr, h1_scr, 16)
    _layer(h1_scr[...], wc2_ref, as2_ref, ad2_ref, mf_ref, wh_scr, es_scr,
           edt_scr, edt2_scr, h2_scr, 32)
    _layer(h2_scr[...], wc3_ref, as3_ref, ad3_ref, mf_ref, wh_scr, es_scr,
           edt_scr, edt2_scr, h3_scr, 64)
    s = jnp.sum(h3_scr[...], axis=0, keepdims=True)
    nrm = jnp.maximum(jnp.sqrt(jnp.sum(s * s)), 1e-12)
    sn = s / nrm
    out_ref[...] = jnp.sum(sn * wdt_ref[...], axis=1, keepdims=True) + bd_ref[...]


def _prep(W, a):
    Hh, fin, f = W.shape
    wcat = jnp.transpose(W, (1, 0, 2)).reshape(fin, Hh * f)
    eye = jnp.eye(Hh, dtype=W.dtype)
    asrc = (a[:, :f][:, :, None] * eye[:, None, :]).reshape(Hh * f, Hh)
    adst = (a[:, f:][:, :, None] * eye[:, None, :]).reshape(Hh * f, Hh)
    return wcat, asrc, adst


@jax.jit
def kernel(x, adj, W1, a1, W2, a2, W3, a3, Wd, bd):
    mf = (adj > 0).astype(jnp.bfloat16)
    wc1, as1, ad1 = _prep(W1, a1)
    wc2, as2, ad2 = _prep(W2, a2)
    wc3, as3, ad3 = _prep(W3, a3)
    wdt = jnp.reshape(Wd, (1, 384))
    bd2 = jnp.reshape(bd, (1, 1))

    out = pl.pallas_call(
        _gnn_kernel,
        out_shape=jax.ShapeDtypeStruct((1, 1), jnp.float32),
        scratch_shapes=[
            pltpu.VMEM((N, H * 128), jnp.bfloat16),
            pltpu.VMEM((N, H), jnp.float32),
            pltpu.VMEM((H, N), jnp.float32),
            pltpu.VMEM((H, N), jnp.float32),
            pltpu.VMEM((N, 96), jnp.float32),
            pltpu.VMEM((N, 192), jnp.float32),
            pltpu.VMEM((N, 384), jnp.float32),
        ],
    )(x, mf, wc1, as1, ad1, wc2, as2, ad2, wc3, as3, ad3, wdt, bd2)
    return jnp.reshape(out, (1,))


# speedup vs baseline: 4.1563x; 1.0160x over previous
import functools

import jax
import jax.numpy as jnp
from jax.experimental import pallas as pl
from jax.experimental.pallas import tpu as pltpu

N = 2048
H = 6
BN = 256
LOG2E = 1.4426950408889634


def _layer(xin, wcat_ref, asrc_ref, adst_ref, mf_ref, whp_scr, es_scr,
           edt_scr, edt2_scr, hout_ref, fout):
    wh = jnp.dot(xin, wcat_ref[...], preferred_element_type=jnp.float32)
    es_scr[...] = jnp.dot(wh, asrc_ref[...], preferred_element_type=jnp.float32)
    e_dst = jnp.dot(wh, adst_ref[...], preferred_element_type=jnp.float32)
    edt = jnp.transpose(e_dst)
    edt_scr[...] = LOG2E * edt
    edt2_scr[...] = (0.2 * LOG2E) * edt
    for h in range(H):
        whp_scr[:, h * 128:h * 128 + fout] = (
            wh[:, h * fout:(h + 1) * fout].astype(jnp.bfloat16))
        whp_scr[:, h * 128 + fout:h * 128 + fout + 1] = jnp.ones(
            (N, 1), jnp.bfloat16)
    edmax = jnp.max(edt, axis=1, keepdims=True)

    def blk(i, _):
        rows = pl.ds(i * BN, BN)
        mfb = mf_ref[rows, :]
        es_blk = es_scr[rows, :]
        for h in range(H):
            esc = es_blk[:, h:h + 1]
            mxc = esc + edmax[h:h + 1, :]
            mx = jnp.maximum(mxc, 0.2 * mxc)
            esm = LOG2E * (esc - mx)
            esm2 = LOG2E * (0.2 * esc - mx)
            e = jnp.maximum(esm + edt_scr[h:h + 1, :],
                            esm2 + edt2_scr[h:h + 1, :])
            p = jnp.exp2(e.astype(jnp.bfloat16)) * mfb
            of = jnp.dot(p, whp_scr[:, h * 128:h * 128 + fout + 1],
                         preferred_element_type=jnp.float32)
            s = jnp.maximum(of[:, fout:fout + 1], 1e-30)
            ob = of[:, :fout] / s
            ob = jnp.where(ob > 0, ob, jnp.exp(ob) - 1.0)
            hout_ref[rows, h * fout:(h + 1) * fout] = ob
        return 0

    jax.lax.fori_loop(0, N // BN, blk, 0)


def _gnn_kernel(x_ref, mf_ref,
                wc1_ref, as1_ref, ad1_ref,
                wc2_ref, as2_ref, ad2_ref,
                wc3_ref, as3_ref, ad3_ref,
                wdt_ref, bd_ref, out_ref,
                wh_scr, es_scr, edt_scr, edt2_scr, h1_scr, h2_scr, h3_scr):
    _layer(x_ref[...], wc1_ref, as1_ref, ad1_ref, mf_ref, wh_scr, es_scr,
           edt_scr, edt2_scr, h1_scr, 16)
    _layer(h1_scr[...], wc2_ref, as2_ref, ad2_ref, mf_ref, wh_scr, es_scr,
           edt_scr, edt2_scr, h2_scr, 32)
    _layer(h2_scr[...], wc3_ref, as3_ref, ad3_ref, mf_ref, wh_scr, es_scr,
           edt_scr, edt2_scr, h3_scr, 64)
    s = jnp.sum(h3_scr[...], axis=0, keepdims=True)
    nrm = jnp.maximum(jnp.sqrt(jnp.sum(s * s)), 1e-12)
    sn = s / nrm
    out_ref[...] = jnp.sum(sn * wdt_ref[...], axis=1, keepdims=True) + bd_ref[...]


def _prep(W, a):
    Hh, fin, f = W.shape
    wcat = jnp.transpose(W, (1, 0, 2)).reshape(fin, Hh * f)
    eye = jnp.eye(Hh, dtype=W.dtype)
    asrc = (a[:, :f][:, :, None] * eye[:, None, :]).reshape(Hh * f, Hh)
    adst = (a[:, f:][:, :, None] * eye[:, None, :]).reshape(Hh * f, Hh)
    return wcat, asrc, adst


@jax.jit
def kernel(x, adj, W1, a1, W2, a2, W3, a3, Wd, bd):
    mf = (adj > 0).astype(jnp.bfloat16)
    wc1, as1, ad1 = _prep(W1, a1)
    wc2, as2, ad2 = _prep(W2, a2)
    wc3, as3, ad3 = _prep(W3, a3)
    wdt = jnp.reshape(Wd, (1, 384))
    bd2 = jnp.reshape(bd, (1, 1))

    out = pl.pallas_call(
        _gnn_kernel,
        out_shape=jax.ShapeDtypeStruct((1, 1), jnp.float32),
        scratch_shapes=[
            pltpu.VMEM((N, H * 128), jnp.bfloat16),
            pltpu.VMEM((N, H), jnp.float32),
            pltpu.VMEM((H, N), jnp.float32),
            pltpu.VMEM((H, N), jnp.float32),
            pltpu.VMEM((N, 96), jnp.float32),
            pltpu.VMEM((N, 192), jnp.float32),
            pltpu.VMEM((N, 384), jnp.float32),
        ],
    )(x, mf, wc1, as1, ad1, wc2, as2, ad2, wc3, as3, ad3, wdt, bd2)
    return jnp.reshape(out, (1,))


# speedup vs baseline: 4.7965x; 1.1540x over previous
import functools

import jax
import jax.numpy as jnp
from jax.experimental import pallas as pl
from jax.experimental.pallas import tpu as pltpu

N = 2048
H = 6
BN = 256
LOG2E = 1.4426950408889634


def _layer(xin, wcat_ref, asrc_ref, adst_ref, mf_ref, whp_scr, es_scr,
           edt_scr, edt2_scr, hout_ref, fout):
    wh = jnp.dot(xin, wcat_ref[...], preferred_element_type=jnp.float32)
    es_scr[...] = jnp.dot(wh, asrc_ref[...], preferred_element_type=jnp.float32)
    e_dst = jnp.dot(wh, adst_ref[...], preferred_element_type=jnp.float32)
    edt = jnp.transpose(e_dst)
    edt_scr[...] = (LOG2E * edt).astype(jnp.bfloat16)
    edt2_scr[...] = ((0.2 * LOG2E) * edt).astype(jnp.bfloat16)
    for h in range(H):
        whp_scr[:, h * 128:h * 128 + fout] = (
            wh[:, h * fout:(h + 1) * fout].astype(jnp.bfloat16))
        whp_scr[:, h * 128 + fout:h * 128 + fout + 1] = jnp.ones(
            (N, 1), jnp.bfloat16)
    edmax = jnp.max(edt, axis=1, keepdims=True)

    def blk(i, _):
        rows = pl.ds(i * BN, BN)
        mfb = mf_ref[rows, :]
        es_blk = es_scr[rows, :]
        for h in range(H):
            esc = es_blk[:, h:h + 1]
            mxc = esc + edmax[h:h + 1, :]
            mx = jnp.maximum(mxc, 0.2 * mxc)
            esm = (LOG2E * (esc - mx)).astype(jnp.bfloat16)
            esm2 = (LOG2E * (0.2 * esc - mx)).astype(jnp.bfloat16)
            e = jnp.maximum(esm + edt_scr[h:h + 1, :],
                            esm2 + edt2_scr[h:h + 1, :])
            p = jnp.exp2(e) * mfb
            of = jnp.dot(p, whp_scr[:, h * 128:h * 128 + fout + 1],
                         preferred_element_type=jnp.float32)
            s = jnp.maximum(of[:, fout:fout + 1], 1e-30)
            ob = of[:, :fout] / s
            ob = jnp.where(ob > 0, ob, jnp.exp(ob) - 1.0)
            hout_ref[rows, h * fout:(h + 1) * fout] = ob
        return 0

    jax.lax.fori_loop(0, N // BN, blk, 0)


def _gnn_kernel(x_ref, mf_ref,
                wc1_ref, as1_ref, ad1_ref,
                wc2_ref, as2_ref, ad2_ref,
                wc3_ref, as3_ref, ad3_ref,
                wdt_ref, bd_ref, out_ref,
                wh_scr, es_scr, edt_scr, edt2_scr, h1_scr, h2_scr, h3_scr):
    _layer(x_ref[...], wc1_ref, as1_ref, ad1_ref, mf_ref, wh_scr, es_scr,
           edt_scr, edt2_scr, h1_scr, 16)
    _layer(h1_scr[...], wc2_ref, as2_ref, ad2_ref, mf_ref, wh_scr, es_scr,
           edt_scr, edt2_scr, h2_scr, 32)
    _layer(h2_scr[...], wc3_ref, as3_ref, ad3_ref, mf_ref, wh_scr, es_scr,
           edt_scr, edt2_scr, h3_scr, 64)
    s = jnp.sum(h3_scr[...], axis=0, keepdims=True)
    nrm = jnp.maximum(jnp.sqrt(jnp.sum(s * s)), 1e-12)
    sn = s / nrm
    out_ref[...] = jnp.sum(sn * wdt_ref[...], axis=1, keepdims=True) + bd_ref[...]


def _prep(W, a):
    Hh, fin, f = W.shape
    wcat = jnp.transpose(W, (1, 0, 2)).reshape(fin, Hh * f)
    eye = jnp.eye(Hh, dtype=W.dtype)
    asrc = (a[:, :f][:, :, None] * eye[:, None, :]).reshape(Hh * f, Hh)
    adst = (a[:, f:][:, :, None] * eye[:, None, :]).reshape(Hh * f, Hh)
    return wcat, asrc, adst


@jax.jit
def kernel(x, adj, W1, a1, W2, a2, W3, a3, Wd, bd):
    mf = (adj > 0).astype(jnp.bfloat16)
    wc1, as1, ad1 = _prep(W1, a1)
    wc2, as2, ad2 = _prep(W2, a2)
    wc3, as3, ad3 = _prep(W3, a3)
    wdt = jnp.reshape(Wd, (1, 384))
    bd2 = jnp.reshape(bd, (1, 1))

    out = pl.pallas_call(
        _gnn_kernel,
        out_shape=jax.ShapeDtypeStruct((1, 1), jnp.float32),
        scratch_shapes=[
            pltpu.VMEM((N, H * 128), jnp.bfloat16),
            pltpu.VMEM((N, H), jnp.float32),
            pltpu.VMEM((H, N), jnp.bfloat16),
            pltpu.VMEM((H, N), jnp.bfloat16),
            pltpu.VMEM((N, 96), jnp.float32),
            pltpu.VMEM((N, 192), jnp.float32),
            pltpu.VMEM((N, 384), jnp.float32),
        ],
    )(x, mf, wc1, as1, ad1, wc2, as2, ad2, wc3, as3, ad3, wdt, bd2)
    return jnp.reshape(out, (1,))


# speedup vs baseline: 4.8180x; 1.0045x over previous
import functools

import jax
import jax.numpy as jnp
from jax.experimental import pallas as pl
from jax.experimental.pallas import tpu as pltpu

N = 2048
H = 6
BN = 256
LOG2E = 1.4426950408889634


def _layer(xin, wcat_ref, asrc_ref, adst_ref, mf_ref, whp_scr, es_scr,
           edt_scr, edt2_scr, hout_ref, fout):
    wh = jnp.dot(xin.astype(jnp.bfloat16), wcat_ref[...],
                 preferred_element_type=jnp.float32)
    wh_bf = wh.astype(jnp.bfloat16)
    es_scr[...] = jnp.dot(wh_bf, asrc_ref[...],
                          preferred_element_type=jnp.float32)
    e_dst = jnp.dot(wh_bf, adst_ref[...],
                    preferred_element_type=jnp.float32)
    edt = jnp.transpose(e_dst)
    edt_scr[...] = (LOG2E * edt).astype(jnp.bfloat16)
    edt2_scr[...] = ((0.2 * LOG2E) * edt).astype(jnp.bfloat16)
    for h in range(H):
        whp_scr[:, h * 128:h * 128 + fout] = wh_bf[:, h * fout:(h + 1) * fout]
        whp_scr[:, h * 128 + fout:h * 128 + fout + 1] = jnp.ones(
            (N, 1), jnp.bfloat16)
    edmax = jnp.max(edt, axis=1, keepdims=True)

    def blk(i, _):
        rows = pl.ds(i * BN, BN)
        mfb = mf_ref[rows, :]
        es_blk = es_scr[rows, :]
        for h in range(H):
            esc = es_blk[:, h:h + 1]
            mxc = esc + edmax[h:h + 1, :]
            mx = jnp.maximum(mxc, 0.2 * mxc)
            esm = (LOG2E * (esc - mx)).astype(jnp.bfloat16)
            esm2 = (LOG2E * (0.2 * esc - mx)).astype(jnp.bfloat16)
            e = jnp.maximum(esm + edt_scr[h:h + 1, :],
                            esm2 + edt2_scr[h:h + 1, :])
            p = jnp.exp2(e) * mfb
            of = jnp.dot(p, whp_scr[:, h * 128:h * 128 + fout + 1],
                         preferred_element_type=jnp.float32)
            s = jnp.maximum(of[:, fout:fout + 1], 1e-30)
            ob = of[:, :fout] / s
            ob = jnp.where(ob > 0, ob, jnp.exp(ob) - 1.0)
            hout_ref[rows, h * fout:(h + 1) * fout] = ob
        return 0

    jax.lax.fori_loop(0, N // BN, blk, 0)


def _gnn_kernel(x_ref, mf_ref,
                wc1_ref, as1_ref, ad1_ref,
                wc2_ref, as2_ref, ad2_ref,
                wc3_ref, as3_ref, ad3_ref,
                wdt_ref, bd_ref, out_ref,
                wh_scr, es_scr, edt_scr, edt2_scr, h1_scr, h2_scr, h3_scr):
    _layer(x_ref[...], wc1_ref, as1_ref, ad1_ref, mf_ref, wh_scr, es_scr,
           edt_scr, edt2_scr, h1_scr, 16)
    _layer(h1_scr[...], wc2_ref, as2_ref, ad2_ref, mf_ref, wh_scr, es_scr,
           edt_scr, edt2_scr, h2_scr, 32)
    _layer(h2_scr[...], wc3_ref, as3_ref, ad3_ref, mf_ref, wh_scr, es_scr,
           edt_scr, edt2_scr, h3_scr, 64)
    s = jnp.sum(h3_scr[...], axis=0, keepdims=True)
    nrm = jnp.maximum(jnp.sqrt(jnp.sum(s * s)), 1e-12)
    sn = s / nrm
    out_ref[...] = jnp.sum(sn * wdt_ref[...], axis=1, keepdims=True) + bd_ref[...]


def _prep(W, a):
    Hh, fin, f = W.shape
    wcat = jnp.transpose(W, (1, 0, 2)).reshape(fin, Hh * f)
    eye = jnp.eye(Hh, dtype=W.dtype)
    asrc = (a[:, :f][:, :, None] * eye[:, None, :]).reshape(Hh * f, Hh)
    adst = (a[:, f:][:, :, None] * eye[:, None, :]).reshape(Hh * f, Hh)
    return (wcat.astype(jnp.bfloat16), asrc.astype(jnp.bfloat16),
            adst.astype(jnp.bfloat16))


@jax.jit
def kernel(x, adj, W1, a1, W2, a2, W3, a3, Wd, bd):
    mf = (adj > 0).astype(jnp.bfloat16)
    wc1, as1, ad1 = _prep(W1, a1)
    wc2, as2, ad2 = _prep(W2, a2)
    wc3, as3, ad3 = _prep(W3, a3)
    wdt = jnp.reshape(Wd, (1, 384))
    bd2 = jnp.reshape(bd, (1, 1))

    out = pl.pallas_call(
        _gnn_kernel,
        out_shape=jax.ShapeDtypeStruct((1, 1), jnp.float32),
        scratch_shapes=[
            pltpu.VMEM((N, H * 128), jnp.bfloat16),
            pltpu.VMEM((N, H), jnp.float32),
            pltpu.VMEM((H, N), jnp.bfloat16),
            pltpu.VMEM((H, N), jnp.bfloat16),
            pltpu.VMEM((N, 96), jnp.float32),
            pltpu.VMEM((N, 192), jnp.float32),
            pltpu.VMEM((N, 384), jnp.float32),
        ],
    )(x, mf, wc1, as1, ad1, wc2, as2, ad2, wc3, as3, ad3, wdt, bd2)
    return jnp.reshape(out, (1,))


# speedup vs baseline: 5.1615x; 1.0713x over previous
import functools

import jax
import jax.numpy as jnp
from jax.experimental import pallas as pl
from jax.experimental.pallas import tpu as pltpu

N = 2048
H = 6
BN = 256
LOG2E = 1.4426950408889634


def _layer(xin, wcat_ref, asrc_ref, adst_ref, mf_ref, whp_scr, es_scr,
           edt_scr, edt2_scr, hout_ref, fout):
    wh = jnp.dot(xin.astype(jnp.bfloat16), wcat_ref[...],
                 preferred_element_type=jnp.float32)
    wh_bf = wh.astype(jnp.bfloat16)
    es_scr[...] = jnp.dot(wh_bf, asrc_ref[...],
                          preferred_element_type=jnp.float32)
    e_dst = jnp.dot(wh_bf, adst_ref[...],
                    preferred_element_type=jnp.float32)
    edt = jnp.transpose(e_dst)
    edt_scr[...] = (LOG2E * edt).astype(jnp.bfloat16)
    edt2_scr[...] = ((0.2 * LOG2E) * edt).astype(jnp.bfloat16)
    for h in range(H):
        whp_scr[:, h * 128:h * 128 + fout] = wh_bf[:, h * fout:(h + 1) * fout]
        whp_scr[:, h * 128 + fout:h * 128 + fout + 1] = jnp.ones(
            (N, 1), jnp.bfloat16)
    edmax = jnp.max(edt, axis=1, keepdims=True)

    def blk(i, _):
        rows = pl.ds(i * BN, BN)
        mfb = mf_ref[rows, :]
        es_blk = es_scr[rows, :]
        for h in range(H):
            esc = es_blk[:, h:h + 1]
            mxc = esc + edmax[h:h + 1, :]
            mx = jnp.maximum(mxc, 0.2 * mxc)
            esm = (LOG2E * (esc - mx)).astype(jnp.bfloat16)
            esm2 = (LOG2E * (0.2 * esc - mx)).astype(jnp.bfloat16)
            e = jnp.maximum(esm + edt_scr[h:h + 1, :],
                            esm2 + edt2_scr[h:h + 1, :])
            p = jnp.exp2(e) * mfb
            of = jnp.dot(p, whp_scr[:, h * 128:h * 128 + fout + 1],
                         preferred_element_type=jnp.float32)
            s = jnp.maximum(of[:, fout:fout + 1], 1e-30)
            ob = of[:, :fout] / s
            ob = jnp.where(ob > 0, ob, jnp.exp(ob) - 1.0)
            hout_ref[rows, h * fout:(h + 1) * fout] = ob
        return 0

    for i in range(N // BN):
        blk(i, 0)


def _gnn_kernel(x_ref, mf_ref,
                wc1_ref, as1_ref, ad1_ref,
                wc2_ref, as2_ref, ad2_ref,
                wc3_ref, as3_ref, ad3_ref,
                wdt_ref, bd_ref, out_ref,
                wh_scr, es_scr, edt_scr, edt2_scr, h1_scr, h2_scr, h3_scr):
    _layer(x_ref[...], wc1_ref, as1_ref, ad1_ref, mf_ref, wh_scr, es_scr,
           edt_scr, edt2_scr, h1_scr, 16)
    _layer(h1_scr[...], wc2_ref, as2_ref, ad2_ref, mf_ref, wh_scr, es_scr,
           edt_scr, edt2_scr, h2_scr, 32)
    _layer(h2_scr[...], wc3_ref, as3_ref, ad3_ref, mf_ref, wh_scr, es_scr,
           edt_scr, edt2_scr, h3_scr, 64)
    s = jnp.sum(h3_scr[...], axis=0, keepdims=True)
    nrm = jnp.maximum(jnp.sqrt(jnp.sum(s * s)), 1e-12)
    sn = s / nrm
    out_ref[...] = jnp.sum(sn * wdt_ref[...], axis=1, keepdims=True) + bd_ref[...]


def _prep(W, a):
    Hh, fin, f = W.shape
    wcat = jnp.transpose(W, (1, 0, 2)).reshape(fin, Hh * f)
    eye = jnp.eye(Hh, dtype=W.dtype)
    asrc = (a[:, :f][:, :, None] * eye[:, None, :]).reshape(Hh * f, Hh)
    adst = (a[:, f:][:, :, None] * eye[:, None, :]).reshape(Hh * f, Hh)
    return (wcat.astype(jnp.bfloat16), asrc.astype(jnp.bfloat16),
            adst.astype(jnp.bfloat16))


@jax.jit
def kernel(x, adj, W1, a1, W2, a2, W3, a3, Wd, bd):
    mf = (adj > 0).astype(jnp.bfloat16)
    wc1, as1, ad1 = _prep(W1, a1)
    wc2, as2, ad2 = _prep(W2, a2)
    wc3, as3, ad3 = _prep(W3, a3)
    wdt = jnp.reshape(Wd, (1, 384))
    bd2 = jnp.reshape(bd, (1, 1))

    out = pl.pallas_call(
        _gnn_kernel,
        out_shape=jax.ShapeDtypeStruct((1, 1), jnp.float32),
        scratch_shapes=[
            pltpu.VMEM((N, H * 128), jnp.bfloat16),
            pltpu.VMEM((N, H), jnp.float32),
            pltpu.VMEM((H, N), jnp.bfloat16),
            pltpu.VMEM((H, N), jnp.bfloat16),
            pltpu.VMEM((N, 96), jnp.float32),
            pltpu.VMEM((N, 192), jnp.float32),
            pltpu.VMEM((N, 384), jnp.float32),
        ],
    )(x, mf, wc1, as1, ad1, wc2, as2, ad2, wc3, as3, ad3, wdt, bd2)
    return jnp.reshape(out, (1,))
